# baseline (speedup 1.0000x reference)
import jax
import jax.numpy as jnp
from jax.experimental import pallas as pl
from jax.experimental.pallas import tpu as pltpu


def _toeplitz_body(r_ref, o_ref, scratch_ref, sem):
    h = r_ref.shape[0]
    two_n = r_ref.shape[1]
    n = two_n // 2

    rows = r_ref[...]
    for d in range(128):
        shift = 127 - d
        scratch_ref[d, :, :] = pltpu.roll(rows, (two_n - shift) % two_n, 1)

    copies = []
    for hh in range(h):
        for c in range(n // 128):
            b0 = n - 128 - 128 * c
            copies.append(pltpu.make_async_copy(
                scratch_ref.at[:, hh, pl.ds(b0, n)],
                o_ref.at[0, hh, pl.ds(128 * c, 128), :],
                sem))
    for cp in copies:
        cp.start()
    for cp in copies:
        cp.wait()


def kernel(relative_position_bias_table, seq_len):
    table = relative_position_bias_table
    h = table.shape[1]
    n = (table.shape[0] + 1) // 2
    r = jnp.flip(table, axis=0).T
    r = jnp.pad(r, ((0, 0), (0, 1)))

    out = pl.pallas_call(
        _toeplitz_body,
        in_specs=[pl.BlockSpec(memory_space=pltpu.MemorySpace.VMEM)],
        out_specs=pl.BlockSpec(memory_space=pltpu.MemorySpace.HBM),
        out_shape=jax.ShapeDtypeStruct((1, h, n, n), table.dtype),
        scratch_shapes=[pltpu.VMEM((128, h, 2 * n), table.dtype),
                        pltpu.SemaphoreType.DMA],
    )(r)
    return out
